# baseline (speedup 1.0000x reference)
import math

import numpy as np
import jax
import jax.numpy as jnp
from jax import lax
from jax.experimental import pallas as pl

HF = WF = 32
NPOS = HF * WF
A = 3
PRE_NMS = 1000
POST_NMS = 10
NMS_THRESH = 0.7
INT_MIN = np.int32(-2147483648)
INT_MAX = np.int32(2147483647)
BIG_IDX = np.int32(1 << 30)


def _anchor_consts():
    size = np.float32(float(min(HF, WF)))
    ratios = np.array([0.5, 1.0, 2.0], np.float32)
    hr = np.sqrt(ratios).astype(np.float32)
    wr = (np.float32(1.0) / hr).astype(np.float32)
    ws = wr * size
    hs = hr * size
    base = np.stack([-ws / 2, -hs / 2, ws / 2, hs / 2], axis=1).astype(np.float32)
    sx = np.arange(WF, dtype=np.float32) * np.float32(16.0)
    sy = np.arange(HF, dtype=np.float32) * np.float32(16.0)
    gy, gx = np.meshgrid(sy, sx, indexing="ij")
    shifts = np.stack([gx, gy, gx, gy], axis=-1).reshape(-1, 4).astype(np.float32)
    anc = (shifts[:, None, :] + base[None, :, :]).astype(np.float32)
    x1, y1, x2, y2 = anc[..., 0], anc[..., 1], anc[..., 2], anc[..., 3]
    wa = (x2 - x1).astype(np.float32)
    ha = (y2 - y1).astype(np.float32)
    cxa = (x1 + np.float32(0.5) * wa).astype(np.float32)
    cya = (y1 + np.float32(0.5) * ha).astype(np.float32)
    return np.stack([wa.T, ha.T, cxa.T, cya.T], axis=0).astype(np.float32)


_ANC = _anchor_consts()
_CLIP_V = math.log(1000.0 / 16)


def _rpn_kernel(p27_ref, w1_ref, whs_ref, wcls_ref, wreg_ref, anc_ref, out_ref):
    f32 = jnp.float32
    i32 = jnp.int32

    patches = p27_ref[...]
    feat = jnp.dot(w1_ref[...], patches, preferred_element_type=f32)
    feat = jnp.maximum(feat, 0.0)

    pidx = lax.broadcasted_iota(i32, (1, NPOS), 1)
    jcol = pidx % WF
    zcol = jnp.zeros((256, 1), f32)

    shifts = []
    for k in range(9):
        dy, dx = k // 3, k % 3
        off = (dy - 1) * WF + (dx - 1)
        if off > 0:
            sh = jnp.concatenate(
                [feat[:, off:], jnp.zeros((256, off), f32)], axis=1)
        elif off < 0:
            sh = jnp.concatenate(
                [jnp.zeros((256, -off), f32), feat[:, :NPOS + off]], axis=1)
        else:
            sh = feat
        if dx == 0:
            sh = jnp.where(jcol == 0, 0.0, sh)
        elif dx == 2:
            sh = jnp.where(jcol == WF - 1, 0.0, sh)
        shifts.append(sh)
    t_acc = jnp.zeros((256, NPOS), f32)
    for k in range(9):
        t_acc = t_acc + jnp.dot(whs_ref[k], shifts[k],
                                preferred_element_type=f32)
    t = jnp.maximum(t_acc, 0.0)

    whd = jnp.concatenate([wcls_ref[...], wreg_ref[...]], axis=0)
    hd = jnp.dot(whd, t, preferred_element_type=f32)
    scores = hd[0:3, :]
    d_x = jnp.concatenate([hd[3:4], hd[7:8], hd[11:12]], axis=0)
    d_y = jnp.concatenate([hd[4:5], hd[8:9], hd[12:13]], axis=0)
    d_w = jnp.concatenate([hd[5:6], hd[9:10], hd[13:14]], axis=0)
    d_h = jnp.concatenate([hd[6:7], hd[10:11], hd[14:15]], axis=0)

    wa = anc_ref[0]
    ha = anc_ref[1]
    cxa = anc_ref[2]
    cya = anc_ref[3]
    dwc = jnp.minimum(d_w, _CLIP_V)
    dhc = jnp.minimum(d_h, _CLIP_V)
    cx = d_x * wa + cxa
    cy = d_y * ha + cya
    pw = jnp.exp(dwc) * wa
    ph = jnp.exp(dhc) * ha
    x1 = jnp.clip(cx - pw / 2, 0.0, 512.0)
    y1 = jnp.clip(cy - ph / 2, 0.0, 512.0)
    x2 = jnp.clip(cx + pw / 2, 0.0, 512.0)
    y2 = jnp.clip(cy + ph / 2, 0.0, 512.0)
    valid = ((x2 - x1) >= 1e-3) & ((y2 - y1) >= 1e-3)

    bits = lax.bitcast_convert_type(scores, i32)
    key = jnp.where(bits >= 0, bits, INT_MIN - bits)
    cnt_pos = jnp.sum((key >= 0).astype(i32))
    cur = jnp.where(cnt_pos >= PRE_NMS, jnp.int32(0), INT_MIN)
    for s in (28, 24, 20, 16, 12, 8, 4, 0):
        cmax = 7 if s == 28 else 15
        cbest = jnp.int32(0)
        for c in range(1, cmax + 1):
            cnt_c = jnp.sum((key >= cur + (c << s)).astype(i32))
            cbest = cbest + (cnt_c >= PRE_NMS).astype(i32)
        cur = cur + (cbest << s)
    in_top = key >= cur

    neg_inf = jnp.float32(-jnp.inf)
    cand = jnp.where(in_top & valid, scores, neg_inf)

    gmax = jnp.max(scores)
    gmask = scores == gmax

    rows_io = lax.broadcasted_iota(i32, (POST_NMS, 4), 0)
    cols_io = lax.broadcasted_iota(i32, (POST_NMS, 4), 1)
    acc = jnp.zeros((POST_NMS, 4), f32)
    suppressed = jnp.zeros((A, NPOS), jnp.bool_)

    for i in range(POST_NMS):
        avail = jnp.where(suppressed, neg_inf, cand)
        m = jnp.max(avail)
        exhausted = m == neg_inf
        smask = (gmask & exhausted) | ((avail == m) & jnp.logical_not(exhausted))
        bx1 = jnp.sum(jnp.where(smask, x1, 0.0))
        by1 = jnp.sum(jnp.where(smask, y1, 0.0))
        bx2 = jnp.sum(jnp.where(smask, x2, 0.0))
        by2 = jnp.sum(jnp.where(smask, y2, 0.0))
        ix1 = jnp.maximum(bx1, x1)
        iy1 = jnp.maximum(by1, y1)
        ix2 = jnp.minimum(bx2, x2)
        iy2 = jnp.minimum(by2, y2)
        inter = jnp.maximum(ix2 - ix1, 0.0) * jnp.maximum(iy2 - iy1, 0.0)
        a1 = (bx2 - bx1) * (by2 - by1)
        a2 = (x2 - x1) * (y2 - y1)
        iou = inter / (a1 + a2 - inter + 1e-9)
        suppressed = suppressed | (iou >= NMS_THRESH) | smask
        row_val = jnp.where(
            cols_io == 0, bx1,
            jnp.where(cols_io == 1, by1, jnp.where(cols_io == 2, bx2, by2)))
        acc = acc + jnp.where(rows_io == i, row_val, 0.0)

    out_ref[0] = acc


def kernel(x, Wb, bb, Wh, bh, Wcls, bcls, Wreg, breg):
    x3 = x[0, 0]
    rows = jnp.stack([x3[:, dy:dy + 497:16, :] for dy in range(3)],
                     axis=1)
    xr = rows.reshape(3, 3, HF, WF, 16)[..., :3]
    p27 = xr.transpose(0, 1, 4, 2, 3).reshape(27, NPOS)

    w1 = Wb.reshape(256, 27)
    whs = Wh.transpose(2, 3, 0, 1).reshape(9, 256, 256)

    return pl.pallas_call(
        _rpn_kernel,
        out_shape=jax.ShapeDtypeStruct((1, POST_NMS, 4), jnp.float32),
    )(p27, w1, whs, Wcls.reshape(3, 256), Wreg.reshape(12, 256),
      jnp.asarray(_ANC))


# speedup vs baseline: 2.8776x; 2.8776x over previous
import math

import numpy as np
import jax
import jax.numpy as jnp
from jax import lax
from jax.experimental import pallas as pl

HF = WF = 32
NPOS = HF * WF
A = 3
PRE_NMS = 1000
POST_NMS = 10
NMS_THRESH = 0.7
INT_MIN = np.int32(-2147483648)


def _anchor_consts():
    size = np.float32(float(min(HF, WF)))
    ratios = np.array([0.5, 1.0, 2.0], np.float32)
    hr = np.sqrt(ratios).astype(np.float32)
    wr = (np.float32(1.0) / hr).astype(np.float32)
    ws = wr * size
    hs = hr * size
    base = np.stack([-ws / 2, -hs / 2, ws / 2, hs / 2], axis=1).astype(np.float32)
    sx = np.arange(WF, dtype=np.float32) * np.float32(16.0)
    sy = np.arange(HF, dtype=np.float32) * np.float32(16.0)
    gy, gx = np.meshgrid(sy, sx, indexing="ij")
    shifts = np.stack([gx, gy, gx, gy], axis=-1).reshape(-1, 4).astype(np.float32)
    anc = (shifts[:, None, :] + base[None, :, :]).astype(np.float32)
    x1, y1, x2, y2 = anc[..., 0], anc[..., 1], anc[..., 2], anc[..., 3]
    wa = (x2 - x1).astype(np.float32)
    ha = (y2 - y1).astype(np.float32)
    cxa = (x1 + np.float32(0.5) * wa).astype(np.float32)
    cya = (y1 + np.float32(0.5) * ha).astype(np.float32)
    return np.stack([wa.T, ha.T, cxa.T, cya.T], axis=0).astype(np.float32)


_ANC = _anchor_consts()
_CLIP_V = math.log(1000.0 / 16)


def _rpn_kernel(p27_ref, w1_ref, whs_ref, wcls_ref, wreg_ref, anc_ref, out_ref):
    f32 = jnp.float32
    i32 = jnp.int32

    patches = p27_ref[...]
    feat = jnp.dot(w1_ref[...], patches, preferred_element_type=f32)
    feat = jnp.maximum(feat, 0.0)

    pidx = lax.broadcasted_iota(i32, (1, NPOS), 1)
    jcol = pidx % WF

    shifts = []
    for k in range(9):
        dy, dx = k // 3, k % 3
        off = (dy - 1) * WF + (dx - 1)
        if off > 0:
            sh = jnp.concatenate(
                [feat[:, off:], jnp.zeros((256, off), f32)], axis=1)
        elif off < 0:
            sh = jnp.concatenate(
                [jnp.zeros((256, -off), f32), feat[:, :NPOS + off]], axis=1)
        else:
            sh = feat
        if dx == 0:
            sh = jnp.where(jcol == 0, 0.0, sh)
        elif dx == 2:
            sh = jnp.where(jcol == WF - 1, 0.0, sh)
        shifts.append(sh)
    t_acc = jnp.zeros((256, NPOS), f32)
    for k in range(9):
        t_acc = t_acc + jnp.dot(whs_ref[k], shifts[k],
                                preferred_element_type=f32)
    t = jnp.maximum(t_acc, 0.0)

    whd = jnp.concatenate([wcls_ref[...], wreg_ref[...]], axis=0)
    hd = jnp.dot(whd, t, preferred_element_type=f32)
    scores = hd[0:3, :]
    d_x = jnp.concatenate([hd[3:4], hd[7:8], hd[11:12]], axis=0)
    d_y = jnp.concatenate([hd[4:5], hd[8:9], hd[12:13]], axis=0)
    d_w = jnp.concatenate([hd[5:6], hd[9:10], hd[13:14]], axis=0)
    d_h = jnp.concatenate([hd[6:7], hd[10:11], hd[14:15]], axis=0)

    wa = anc_ref[0]
    ha = anc_ref[1]
    cxa = anc_ref[2]
    cya = anc_ref[3]
    dwc = jnp.minimum(d_w, _CLIP_V)
    dhc = jnp.minimum(d_h, _CLIP_V)
    cx = d_x * wa + cxa
    cy = d_y * ha + cya
    pw = jnp.exp(dwc) * wa
    ph = jnp.exp(dhc) * ha
    x1 = jnp.clip(cx - pw / 2, 0.0, 512.0)
    y1 = jnp.clip(cy - ph / 2, 0.0, 512.0)
    x2 = jnp.clip(cx + pw / 2, 0.0, 512.0)
    y2 = jnp.clip(cy + ph / 2, 0.0, 512.0)
    valid = ((x2 - x1) >= 1e-3) & ((y2 - y1) >= 1e-3)

    bits = lax.bitcast_convert_type(scores, i32)
    key = jnp.where(bits >= 0, bits, INT_MIN - bits)
    cnt_pos = jnp.sum((key >= 0).astype(i32))
    cur = jnp.where(cnt_pos >= PRE_NMS, jnp.int32(0), INT_MIN)
    for s in (28, 24, 20, 16, 12, 8, 4, 0):
        cmax = 7 if s == 28 else 15
        cbest = jnp.int32(0)
        for c in range(1, cmax + 1):
            cnt_c = jnp.sum((key >= cur + (c << s)).astype(i32))
            cbest = cbest + (cnt_c >= PRE_NMS).astype(i32)
        cur = cur + (cbest << s)
    in_top = key >= cur

    neg_inf = jnp.float32(-jnp.inf)
    cand = jnp.where(in_top & valid, scores, neg_inf)

    gmax = jnp.max(scores)
    gmask = scores == gmax

    rows_io = lax.broadcasted_iota(i32, (POST_NMS, 4), 0)
    cols_io = lax.broadcasted_iota(i32, (POST_NMS, 4), 1)
    acc = jnp.zeros((POST_NMS, 4), f32)
    suppressed = jnp.zeros((A, NPOS), jnp.bool_)

    for i in range(POST_NMS):
        avail = jnp.where(suppressed, neg_inf, cand)
        m = jnp.max(avail)
        exhausted = m == neg_inf
        smask = (gmask & exhausted) | ((avail == m) & jnp.logical_not(exhausted))
        bx1 = jnp.sum(jnp.where(smask, x1, 0.0))
        by1 = jnp.sum(jnp.where(smask, y1, 0.0))
        bx2 = jnp.sum(jnp.where(smask, x2, 0.0))
        by2 = jnp.sum(jnp.where(smask, y2, 0.0))
        ix1 = jnp.maximum(bx1, x1)
        iy1 = jnp.maximum(by1, y1)
        ix2 = jnp.minimum(bx2, x2)
        iy2 = jnp.minimum(by2, y2)
        inter = jnp.maximum(ix2 - ix1, 0.0) * jnp.maximum(iy2 - iy1, 0.0)
        a1 = (bx2 - bx1) * (by2 - by1)
        a2 = (x2 - x1) * (y2 - y1)
        iou = inter / (a1 + a2 - inter + 1e-9)
        suppressed = suppressed | (iou >= NMS_THRESH) | smask
        row_val = jnp.where(
            cols_io == 0, bx1,
            jnp.where(cols_io == 1, by1, jnp.where(cols_io == 2, bx2, by2)))
        acc = acc + jnp.where(rows_io == i, row_val, 0.0)

    out_ref[0] = acc


def kernel(x, Wb, bb, Wh, bh, Wcls, bcls, Wreg, breg):
    x3 = x[0, 0]
    xr = x3.reshape(3, HF, 16, 512)[:, :, :3, :]
    xr = xr.reshape(3, HF, 3, WF, 16)[..., :3]
    p27 = xr.transpose(0, 2, 4, 1, 3).reshape(27, NPOS)

    w1 = Wb.reshape(256, 27)
    whs = Wh.transpose(2, 3, 0, 1).reshape(9, 256, 256)

    return pl.pallas_call(
        _rpn_kernel,
        out_shape=jax.ShapeDtypeStruct((1, POST_NMS, 4), jnp.float32),
    )(p27, w1, whs, Wcls.reshape(3, 256), Wreg.reshape(12, 256),
      jnp.asarray(_ANC))
